# Initial kernel scaffold; baseline (speedup 1.0000x reference)
import jax
import jax.numpy as jnp
from jax.experimental import pallas as pl


def kernel(x, position_ids, cos_cached, sin_cached):
    raise NotImplementedError("write your pallas kernel here")


# speedup vs baseline: 1.5250x; 1.5250x over previous
import jax
import jax.numpy as jnp
from jax import lax
from jax.experimental import pallas as pl
from jax.experimental.pallas import tpu as pltpu
from jax.experimental.pallas import tpu_sc as plsc

DIM = 128
CHUNK = 128
NC = 2
NS = 16
NW = NC * NS


def _gather_body(idx_hbm, cos_hbm, sin_hbm, cos_out, sin_out,
                 idx_v, cbuf, sbuf, csem, ssem):
    n_chunks = idx_v.shape[0]
    wid = lax.axis_index("s") * NC + lax.axis_index("c")
    pltpu.sync_copy(idx_hbm.at[wid], idx_v)
    for c in range(n_chunks):
        row0 = wid * (n_chunks * CHUNK) + c * CHUNK
        ccp = pltpu.async_copy(cos_hbm.at[idx_v.at[c]], cbuf.at[c % 2], csem)
        scp = pltpu.async_copy(sin_hbm.at[idx_v.at[c]], sbuf.at[c % 2], ssem)
        ccp.wait()
        pltpu.sync_copy(cbuf.at[c % 2], cos_out.at[pl.ds(row0, CHUNK)])
        scp.wait()
        pltpu.sync_copy(sbuf.at[c % 2], sin_out.at[pl.ds(row0, CHUNK)])


def kernel(x, position_ids, cos_cached, sin_cached):
    B, S = position_ids.shape
    total = B * S
    per_w = total // NW
    n_chunks = per_w // CHUNK
    idx = position_ids.reshape(NW, n_chunks, CHUNK)

    mesh = plsc.VectorSubcoreMesh(core_axis_name="c", subcore_axis_name="s")
    out_t = (
        jax.ShapeDtypeStruct((total, DIM), jnp.float32),
        jax.ShapeDtypeStruct((total, DIM), jnp.float32),
    )
    fn = pl.kernel(
        _gather_body,
        out_type=out_t,
        mesh=mesh,
        scratch_types=[
            pltpu.VMEM((n_chunks, CHUNK), jnp.int32),
            pltpu.VMEM((2, CHUNK, DIM), jnp.float32),
            pltpu.VMEM((2, CHUNK, DIM), jnp.float32),
            pltpu.SemaphoreType.DMA,
            pltpu.SemaphoreType.DMA,
        ],
    )
    cos_flat, sin_flat = fn(idx, cos_cached, sin_cached)
    return (cos_flat.reshape(B, S, DIM), sin_flat.reshape(B, S, DIM))


# speedup vs baseline: 1.6081x; 1.0544x over previous
import jax
import jax.numpy as jnp
from jax import lax
from jax.experimental import pallas as pl
from jax.experimental.pallas import tpu as pltpu
from jax.experimental.pallas import tpu_sc as plsc

DIM = 128
CHUNK = 128
NC = 2
NS = 16
NW = NC * NS


def _gather_body(idx_hbm, cos_hbm, sin_hbm, cos_out, sin_out,
                 idx_v, cbuf, sbuf, cg0, cg1, cw0, cw1, sg0, sg1, sw0, sw1):
    cg, cw, sg, sw = (cg0, cg1), (cw0, cw1), (sg0, sg1), (sw0, sw1)
    n = idx_v.shape[0]
    wid = lax.axis_index("s") * NC + lax.axis_index("c")
    base = wid * (n * CHUNK)
    pltpu.sync_copy(idx_hbm.at[wid], idx_v)

    gc, gs, wc, ws = {}, {}, {}, {}

    def issue_gather(c):
        slot = c % 2
        gc[c] = pltpu.async_copy(cos_hbm.at[idx_v.at[c]], cbuf.at[slot], cg[slot])
        gs[c] = pltpu.async_copy(sin_hbm.at[idx_v.at[c]], sbuf.at[slot], sg[slot])

    issue_gather(0)
    if n > 1:
        issue_gather(1)
    for c in range(n):
        slot = c % 2
        row0 = base + c * CHUNK
        gc[c].wait()
        wc[c] = pltpu.async_copy(cbuf.at[slot], cos_out.at[pl.ds(row0, CHUNK)], cw[slot])
        gs[c].wait()
        ws[c] = pltpu.async_copy(sbuf.at[slot], sin_out.at[pl.ds(row0, CHUNK)], sw[slot])
        if c + 2 < n:
            wc[c].wait()
            ws[c].wait()
            issue_gather(c + 2)
    for c in range(max(0, n - 2), n):
        wc[c].wait()
        ws[c].wait()


def kernel(x, position_ids, cos_cached, sin_cached):
    B, S = position_ids.shape
    total = B * S
    per_w = total // NW
    n_chunks = per_w // CHUNK
    idx = position_ids.reshape(NW, n_chunks, CHUNK)

    mesh = plsc.VectorSubcoreMesh(core_axis_name="c", subcore_axis_name="s")
    out_t = (
        jax.ShapeDtypeStruct((total, DIM), jnp.float32),
        jax.ShapeDtypeStruct((total, DIM), jnp.float32),
    )
    fn = pl.kernel(
        _gather_body,
        out_type=out_t,
        mesh=mesh,
        scratch_types=[
            pltpu.VMEM((n_chunks, CHUNK), jnp.int32),
            pltpu.VMEM((2, CHUNK, DIM), jnp.float32),
            pltpu.VMEM((2, CHUNK, DIM), jnp.float32),
        ] + [pltpu.SemaphoreType.DMA] * 8,
    )
    cos_flat, sin_flat = fn(idx, cos_cached, sin_cached)
    return (cos_flat.reshape(B, S, DIM), sin_flat.reshape(B, S, DIM))


# speedup vs baseline: 1.7014x; 1.0581x over previous
import jax
import jax.numpy as jnp
from jax import lax
from jax.experimental import pallas as pl
from jax.experimental.pallas import tpu as pltpu
from jax.experimental.pallas import tpu_sc as plsc

DIM = 128
CHUNK = 128
NC = 2
NS = 16
NW = NC * NS


NBUF = 3


def _gather_body(idx_hbm, cos_hbm, sin_hbm, cos_out, sin_out,
                 idx_v, cbuf, sbuf, *sems):
    cg, cw, sg, sw = (sems[0:NBUF], sems[NBUF:2 * NBUF],
                      sems[2 * NBUF:3 * NBUF], sems[3 * NBUF:4 * NBUF])
    n = idx_v.shape[0]
    wid = lax.axis_index("s") * NC + lax.axis_index("c")
    base = wid * (n * CHUNK)
    pltpu.sync_copy(idx_hbm.at[wid], idx_v)

    gc, gs, wc, ws = {}, {}, {}, {}

    def issue_gather(c):
        slot = c % NBUF
        gc[c] = pltpu.async_copy(cos_hbm.at[idx_v.at[c]], cbuf.at[slot], cg[slot])
        gs[c] = pltpu.async_copy(sin_hbm.at[idx_v.at[c]], sbuf.at[slot], sg[slot])

    for c in range(min(NBUF, n)):
        issue_gather(c)
    for c in range(n):
        slot = c % NBUF
        row0 = base + c * CHUNK
        gc[c].wait()
        wc[c] = pltpu.async_copy(cbuf.at[slot], cos_out.at[pl.ds(row0, CHUNK)], cw[slot])
        gs[c].wait()
        ws[c] = pltpu.async_copy(sbuf.at[slot], sin_out.at[pl.ds(row0, CHUNK)], sw[slot])
        if c + NBUF < n:
            wc[c].wait()
            ws[c].wait()
            issue_gather(c + NBUF)
    for c in range(max(0, n - NBUF), n):
        wc[c].wait()
        ws[c].wait()


def kernel(x, position_ids, cos_cached, sin_cached):
    B, S = position_ids.shape
    total = B * S
    per_w = total // NW
    n_chunks = per_w // CHUNK
    idx = position_ids.reshape(NW, n_chunks, CHUNK)

    mesh = plsc.VectorSubcoreMesh(core_axis_name="c", subcore_axis_name="s")
    out_t = (
        jax.ShapeDtypeStruct((total, DIM), jnp.float32),
        jax.ShapeDtypeStruct((total, DIM), jnp.float32),
    )
    fn = pl.kernel(
        _gather_body,
        out_type=out_t,
        mesh=mesh,
        scratch_types=[
            pltpu.VMEM((n_chunks, CHUNK), jnp.int32),
            pltpu.VMEM((NBUF, CHUNK, DIM), jnp.float32),
            pltpu.VMEM((NBUF, CHUNK, DIM), jnp.float32),
        ] + [pltpu.SemaphoreType.DMA] * (4 * NBUF),
    )
    cos_flat, sin_flat = fn(idx, cos_cached, sin_cached)
    return (cos_flat.reshape(B, S, DIM), sin_flat.reshape(B, S, DIM))
